# baseline (speedup 1.0000x reference)
import functools

import jax
import jax.numpy as jnp
from jax import lax
from jax.experimental import pallas as pl
from jax.experimental.pallas import tpu as pltpu
from jax.experimental.pallas import tpu_sc as plsc

N_NODES = 10000
NFEAT = 128
NHID = 512
NCLASS = 64
N_EDGES = 320000

NC = 2
NS = 16
CHUNK = 128
NCHUNK = 80
EDGE_PAD = NC * NS * NCHUNK * CHUNK
ACC_ROWS = 10240
ROWS_PER_TILE = ACC_ROWS // NS
ROW_BLK = 1280
N_BLOCKS = ACC_ROWS // ROW_BLK


def _sc_mesh():
    return plsc.VectorSubcoreMesh(core_axis_name="c", subcore_axis_name="s")


EDGES_PER_TILE = NCHUNK * CHUNK


def _deg_body(dst3f, zeros1, degp, didx1, hist):
    c = lax.axis_index("c")
    s = lax.axis_index("s")
    pltpu.sync_copy(zeros1, hist)
    pltpu.sync_copy(dst3f.at[c, s], didx1)
    ones_v = jnp.ones((16,), jnp.float32)

    def body(i, _):
        idx = didx1[pl.ds(i * 16, 16)]
        plsc.addupdate_scatter(hist, [idx], ones_v)
        return ()

    lax.fori_loop(0, EDGES_PER_TILE // 16, body, (), unroll=4)
    pltpu.sync_copy(hist, degp.at[c, s])


def _deg_pass(dst3f, zeros1):
    k = pl.kernel(
        _deg_body,
        out_type=jax.ShapeDtypeStruct((NC, NS, ACC_ROWS), jnp.float32),
        mesh=_sc_mesh(),
        scratch_types=[
            pltpu.VMEM((EDGES_PER_TILE,), jnp.int32),
            pltpu.VMEM((ACC_ROWS,), jnp.float32),
        ],
        compiler_params=pltpu.CompilerParams(needs_layout_passes=False),
    )
    return k(dst3f, zeros1)


W64 = NCLASS


NBUF = 4


def _make_agg_body(nslab):
    def body(*args):
        src4, dst4 = args[0], args[1]
        feats = args[2:2 + nslab]
        zerosw = args[2 + nslab]
        outs = args[3 + nslab:3 + 2 * nslab]
        rest = args[3 + 2 * nslab:]
        sidx, didx = rest[0], rest[1]
        bufs = rest[2:2 + NBUF]
        acc = rest[2 + NBUF]
        gsems = rest[3 + NBUF:3 + 2 * NBUF]
        ssems = rest[3 + 2 * NBUF:3 + 3 * NBUF]
        c = lax.axis_index("c")
        s = lax.axis_index("s")
        row0 = s * ROWS_PER_TILE
        pltpu.sync_copy(src4.at[c, s], sidx)
        pltpu.sync_copy(dst4.at[c, s], didx)

        def wait_gather(feat, buf, sem):
            pltpu.make_async_copy(feat.at[pl.ds(0, CHUNK)], buf, sem).wait()

        def wait_scatter(buf, sem):
            pltpu.make_async_copy(buf, acc.at[pl.ds(0, CHUNK)], sem).wait()

        for feat, outp in zip(feats, outs):
            pltpu.sync_copy(zerosw.at[pl.ds(row0, ROWS_PER_TILE)],
                            acc.at[pl.ds(row0, ROWS_PER_TILE)])
            plsc.subcore_barrier()

            for b in range(NBUF):
                pltpu.async_copy(feat.at[sidx.at[b]], bufs[b], gsems[b])

            def body2(j, _):
                for b in range(NBUF):
                    wait_gather(feat, bufs[b], gsems[b])
                    pltpu.async_copy(bufs[b], acc.at[didx.at[j + b]],
                                     ssems[b], add=True)
                for b in range(NBUF):
                    wait_scatter(bufs[b], ssems[b])

                    @pl.when(j + b + NBUF < NCHUNK)
                    def _():
                        pltpu.async_copy(feat.at[sidx.at[j + b + NBUF]],
                                         bufs[b], gsems[b])

                return ()

            lax.fori_loop(0, NCHUNK // NBUF,
                          lambda i, cr: body2(i * NBUF, cr), (), unroll=1)
            plsc.subcore_barrier()
            pltpu.sync_copy(acc.at[pl.ds(row0, ROWS_PER_TILE)],
                            outp.at[c, pl.ds(row0, ROWS_PER_TILE)])
    return body


def _agg_pass(src4, dst4, feats, zerosw):
    nslab = len(feats)
    k = pl.kernel(
        _make_agg_body(nslab),
        out_type=[jax.ShapeDtypeStruct((NC, ACC_ROWS, W64), jnp.float32)
                  for _ in range(nslab)],
        mesh=_sc_mesh(),
        scratch_types=[
            pltpu.VMEM((NCHUNK, CHUNK), jnp.int32),
            pltpu.VMEM((NCHUNK, CHUNK), jnp.int32),
            *[pltpu.VMEM((CHUNK, W64), jnp.float32) for _ in range(NBUF)],
            pltpu.VMEM_SHARED((ACC_ROWS, W64), jnp.float32),
            *[pltpu.SemaphoreType.DMA for _ in range(2 * NBUF)],
        ],
        compiler_params=pltpu.CompilerParams(use_tc_tiling_on_sc=False),
    )
    return k(src4, dst4, *feats, zerosw)


def _prescale_body(degp_ref, x_ref, xpl_ref, xpr_ref, dinvb_ref):
    dp = degp_ref[...]
    deg = 1.0 + jnp.sum(dp, axis=1, keepdims=True)
    dinv = lax.rsqrt(deg)
    dinvb = jnp.broadcast_to(dinv, (ROW_BLK, NFEAT))
    dinvb_ref[...] = dinvb
    xp = x_ref[...] * dinvb
    xpl_ref[...] = xp[:, :W64]
    xpr_ref[...] = xp[:, W64:]


def _tc_prescale(degp_t, x):
    return pl.pallas_call(
        _prescale_body,
        grid=(N_BLOCKS,),
        in_specs=[
            pl.BlockSpec((ROW_BLK, NC * NS), lambda i: (i, 0)),
            pl.BlockSpec((ROW_BLK, NFEAT), lambda i: (i, 0)),
        ],
        out_specs=[
            pl.BlockSpec((ROW_BLK, W64), lambda i: (i, 0)),
            pl.BlockSpec((ROW_BLK, W64), lambda i: (i, 0)),
            pl.BlockSpec((ROW_BLK, NFEAT), lambda i: (i, 0)),
        ],
        out_shape=[
            jax.ShapeDtypeStruct((ACC_ROWS, W64), jnp.float32),
            jax.ShapeDtypeStruct((ACC_ROWS, W64), jnp.float32),
            jax.ShapeDtypeStruct((ACC_ROWS, NFEAT), jnp.float32),
        ],
    )(degp_t, x)


def _mid_body(aggl_ref, aggr_ref, xpl_ref, xpr_ref, dinvb_ref,
              w1_ref, b1_ref, w2_ref, y_ref):
    al = aggl_ref[...]
    ar = aggr_ref[...]
    zl = al[0] + al[1] + xpl_ref[...]
    zr = ar[0] + ar[1] + xpr_ref[...]
    dinvb = dinvb_ref[...]
    z1 = dinvb * jnp.concatenate([zl, zr], axis=1)
    h = jnp.dot(z1, w1_ref[...], preferred_element_type=jnp.float32)
    h = jnp.maximum(h + b1_ref[...], 0.0)
    y = jnp.dot(h, w2_ref[...], preferred_element_type=jnp.float32)
    y_ref[...] = y * dinvb[:, :NCLASS]


def _tc_mid(aggl, aggr, xpl, xpr, dinvb, W1, b1r, W2):
    return pl.pallas_call(
        _mid_body,
        grid=(N_BLOCKS,),
        in_specs=[
            pl.BlockSpec((NC, ROW_BLK, W64), lambda i: (0, i, 0)),
            pl.BlockSpec((NC, ROW_BLK, W64), lambda i: (0, i, 0)),
            pl.BlockSpec((ROW_BLK, W64), lambda i: (i, 0)),
            pl.BlockSpec((ROW_BLK, W64), lambda i: (i, 0)),
            pl.BlockSpec((ROW_BLK, NFEAT), lambda i: (i, 0)),
            pl.BlockSpec((NFEAT, NHID), lambda i: (0, 0)),
            pl.BlockSpec((1, NHID), lambda i: (0, 0)),
            pl.BlockSpec((NHID, NCLASS), lambda i: (0, 0)),
        ],
        out_specs=pl.BlockSpec((ROW_BLK, NCLASS), lambda i: (i, 0)),
        out_shape=jax.ShapeDtypeStruct((ACC_ROWS, NCLASS), jnp.float32),
    )(aggl, aggr, xpl, xpr, dinvb, W1, b1r, W2)


def _final_body(aggp_ref, y_ref, dinvb_ref, b2_ref, out_ref):
    ap = aggp_ref[...]
    dinv = dinvb_ref[...][:, :NCLASS]
    z = dinv * (ap[0] + ap[1] + y_ref[...]) + b2_ref[...]
    zs = z * 5.0
    m = jnp.max(zs, axis=1, keepdims=True)
    e = jnp.exp(zs - m)
    lse = jnp.log(jnp.sum(e, axis=1, keepdims=True))
    out_ref[...] = (zs - m) - lse


def _tc_final(aggp, y, dinvb, b2r):
    return pl.pallas_call(
        _final_body,
        grid=(N_BLOCKS,),
        in_specs=[
            pl.BlockSpec((NC, ROW_BLK, NCLASS), lambda i: (0, i, 0)),
            pl.BlockSpec((ROW_BLK, NCLASS), lambda i: (i, 0)),
            pl.BlockSpec((ROW_BLK, NFEAT), lambda i: (i, 0)),
            pl.BlockSpec((1, NCLASS), lambda i: (0, 0)),
        ],
        out_specs=pl.BlockSpec((ROW_BLK, NCLASS), lambda i: (i, 0)),
        out_shape=jax.ShapeDtypeStruct((ACC_ROWS, NCLASS), jnp.float32),
    )(aggp, y, dinvb, b2r)


def kernel(x, edge_index, W1, b1, W2, b2):
    src = edge_index[0].astype(jnp.int32)
    dst = edge_index[1].astype(jnp.int32)
    npad = EDGE_PAD - N_EDGES
    srcp = jnp.concatenate([src, jnp.zeros((npad,), jnp.int32)])
    dstp = jnp.concatenate([dst, jnp.full((npad,), N_NODES, jnp.int32)])
    src4 = srcp.reshape(NC, NS, NCHUNK, CHUNK)
    dst4 = dstp.reshape(NC, NS, NCHUNK, CHUNK)
    dst3f = dstp.reshape(NC, NS, NCHUNK * CHUNK)

    zeros1 = jnp.zeros((ACC_ROWS,), jnp.float32)
    zeros64 = jnp.zeros((ACC_ROWS, W64), jnp.float32)

    xpad = jnp.zeros((ACC_ROWS, NFEAT), jnp.float32).at[:N_NODES].set(x)

    degp = _deg_pass(dst3f, zeros1)
    degp_t = degp.reshape(NC * NS, ACC_ROWS).T
    xpl, xpr, dinvb = _tc_prescale(degp_t, xpad)
    aggl, aggr = _agg_pass(src4, dst4, [xpl, xpr], zeros64)
    y = _tc_mid(aggl, aggr, xpl, xpr, dinvb, W1, b1.reshape(1, NHID), W2)
    (agg2,) = _agg_pass(src4, dst4, [y], zeros64)
    out = _tc_final(agg2, y, dinvb, b2.reshape(1, NCLASS))
    return out[:N_NODES]


# speedup vs baseline: 1.0639x; 1.0639x over previous
import functools

import jax
import jax.numpy as jnp
from jax import lax
from jax.experimental import pallas as pl
from jax.experimental.pallas import tpu as pltpu
from jax.experimental.pallas import tpu_sc as plsc

N_NODES = 10000
NFEAT = 128
NHID = 512
NCLASS = 64
N_EDGES = 320000

NC = 2
NS = 16
CHUNK = 128
NCHUNK = 80
EDGE_PAD = NC * NS * NCHUNK * CHUNK
ACC_ROWS = 10240
ROWS_PER_TILE = ACC_ROWS // NS
ROW_BLK = 1280
N_BLOCKS = ACC_ROWS // ROW_BLK


def _sc_mesh():
    return plsc.VectorSubcoreMesh(core_axis_name="c", subcore_axis_name="s")


EDGES_PER_TILE = NCHUNK * CHUNK


def _deg_body(dst3f, zeros1, degp, didx1, hist):
    c = lax.axis_index("c")
    s = lax.axis_index("s")
    pltpu.sync_copy(zeros1, hist)
    pltpu.sync_copy(dst3f.at[c, s], didx1)
    ones_v = jnp.ones((16,), jnp.float32)

    def body(i, _):
        idx = didx1[pl.ds(i * 16, 16)]
        plsc.addupdate_scatter(hist, [idx], ones_v)
        return ()

    lax.fori_loop(0, EDGES_PER_TILE // 16, body, (), unroll=4)
    pltpu.sync_copy(hist, degp.at[c, s])


def _deg_pass(dst3f, zeros1):
    k = pl.kernel(
        _deg_body,
        out_type=jax.ShapeDtypeStruct((NC, NS, ACC_ROWS), jnp.float32),
        mesh=_sc_mesh(),
        scratch_types=[
            pltpu.VMEM((EDGES_PER_TILE,), jnp.int32),
            pltpu.VMEM((ACC_ROWS,), jnp.float32),
        ],
        compiler_params=pltpu.CompilerParams(needs_layout_passes=False),
    )
    return k(dst3f, zeros1)


W64 = NCLASS


NBUF = 4
N0 = 120
N1 = 40
N0TOT = NS * N0


def _make_agg_body(nslab):
    def body(*args):
        srcf, dstf = args[0], args[1]
        feats = args[2:2 + nslab]
        zerosw = args[2 + nslab]
        outs = args[3 + nslab:3 + 2 * nslab]
        rest = args[3 + 2 * nslab:]
        sidx, didx = rest[0], rest[1]
        bufs = rest[2:2 + NBUF]
        acc = rest[2 + NBUF]
        gsems = rest[3 + NBUF:3 + 2 * NBUF]
        ssems = rest[3 + 2 * NBUF:3 + 3 * NBUF]
        c = lax.axis_index("c")
        s = lax.axis_index("s")
        row0 = s * ROWS_PER_TILE

        @pl.when(c == 0)
        def _():
            pltpu.sync_copy(srcf.at[pl.ds(s * N0, N0)], sidx.at[pl.ds(0, N0)])
            pltpu.sync_copy(dstf.at[pl.ds(s * N0, N0)], didx.at[pl.ds(0, N0)])

        @pl.when(c == 1)
        def _():
            pltpu.sync_copy(srcf.at[pl.ds(N0TOT + s * N1, N1)],
                            sidx.at[pl.ds(0, N1)])
            pltpu.sync_copy(dstf.at[pl.ds(N0TOT + s * N1, N1)],
                            didx.at[pl.ds(0, N1)])

        def wait_gather(feat, buf, sem):
            pltpu.make_async_copy(feat.at[pl.ds(0, CHUNK)], buf, sem).wait()

        def wait_scatter(buf, sem):
            pltpu.make_async_copy(buf, acc.at[pl.ds(0, CHUNK)], sem).wait()

        def pipeline(feat, nchunk):
            for b in range(NBUF):
                pltpu.async_copy(feat.at[sidx.at[b]], bufs[b], gsems[b])

            def body2(j, _):
                for b in range(NBUF):
                    wait_gather(feat, bufs[b], gsems[b])
                    pltpu.async_copy(bufs[b], acc.at[didx.at[j + b]],
                                     ssems[b], add=True)
                for b in range(NBUF):
                    wait_scatter(bufs[b], ssems[b])

                    @pl.when(j + b + NBUF < nchunk)
                    def _():
                        pltpu.async_copy(feat.at[sidx.at[j + b + NBUF]],
                                         bufs[b], gsems[b])

                return ()

            lax.fori_loop(0, nchunk // NBUF,
                          lambda i, cr: body2(i * NBUF, cr), (), unroll=1)

        for feat, outp in zip(feats, outs):
            pltpu.sync_copy(zerosw.at[pl.ds(row0, ROWS_PER_TILE)],
                            acc.at[pl.ds(row0, ROWS_PER_TILE)])
            plsc.subcore_barrier()

            @pl.when(c == 0)
            def _():
                pipeline(feat, N0)

            @pl.when(c == 1)
            def _():
                pipeline(feat, N1)

            plsc.subcore_barrier()
            pltpu.sync_copy(acc.at[pl.ds(row0, ROWS_PER_TILE)],
                            outp.at[c, pl.ds(row0, ROWS_PER_TILE)])
    return body


def _agg_pass(srcf, dstf, feats, zerosw):
    nslab = len(feats)
    k = pl.kernel(
        _make_agg_body(nslab),
        out_type=[jax.ShapeDtypeStruct((NC, ACC_ROWS, W64), jnp.float32)
                  for _ in range(nslab)],
        mesh=_sc_mesh(),
        scratch_types=[
            pltpu.VMEM((N0, CHUNK), jnp.int32),
            pltpu.VMEM((N0, CHUNK), jnp.int32),
            *[pltpu.VMEM((CHUNK, W64), jnp.float32) for _ in range(NBUF)],
            pltpu.VMEM_SHARED((ACC_ROWS, W64), jnp.float32),
            *[pltpu.SemaphoreType.DMA for _ in range(2 * NBUF)],
        ],
        compiler_params=pltpu.CompilerParams(use_tc_tiling_on_sc=False),
    )
    return k(srcf, dstf, *feats, zerosw)


def _prescale_body(degp_ref, x_ref, xpl_ref, xpr_ref, dinvb_ref):
    dp = degp_ref[...]
    deg = 1.0 + jnp.sum(dp, axis=1, keepdims=True)
    dinv = lax.rsqrt(deg)
    dinvb = jnp.broadcast_to(dinv, (ROW_BLK, NFEAT))
    dinvb_ref[...] = dinvb
    xp = x_ref[...] * dinvb
    xpl_ref[...] = xp[:, :W64]
    xpr_ref[...] = xp[:, W64:]


def _tc_prescale(degp_t, x):
    return pl.pallas_call(
        _prescale_body,
        grid=(N_BLOCKS,),
        in_specs=[
            pl.BlockSpec((ROW_BLK, NC * NS), lambda i: (i, 0)),
            pl.BlockSpec((ROW_BLK, NFEAT), lambda i: (i, 0)),
        ],
        out_specs=[
            pl.BlockSpec((ROW_BLK, W64), lambda i: (i, 0)),
            pl.BlockSpec((ROW_BLK, W64), lambda i: (i, 0)),
            pl.BlockSpec((ROW_BLK, NFEAT), lambda i: (i, 0)),
        ],
        out_shape=[
            jax.ShapeDtypeStruct((ACC_ROWS, W64), jnp.float32),
            jax.ShapeDtypeStruct((ACC_ROWS, W64), jnp.float32),
            jax.ShapeDtypeStruct((ACC_ROWS, NFEAT), jnp.float32),
        ],
    )(degp_t, x)


def _mid_body(aggl_ref, aggr_ref, xpl_ref, xpr_ref, dinvb_ref,
              w1_ref, b1_ref, w2_ref, y_ref):
    al = aggl_ref[...]
    ar = aggr_ref[...]
    zl = al[0] + al[1] + xpl_ref[...]
    zr = ar[0] + ar[1] + xpr_ref[...]
    dinvb = dinvb_ref[...]
    z1 = dinvb * jnp.concatenate([zl, zr], axis=1)
    h = jnp.dot(z1, w1_ref[...], preferred_element_type=jnp.float32)
    h = jnp.maximum(h + b1_ref[...], 0.0)
    y = jnp.dot(h, w2_ref[...], preferred_element_type=jnp.float32)
    y_ref[...] = y * dinvb[:, :NCLASS]


def _tc_mid(aggl, aggr, xpl, xpr, dinvb, W1, b1r, W2):
    return pl.pallas_call(
        _mid_body,
        grid=(N_BLOCKS,),
        in_specs=[
            pl.BlockSpec((NC, ROW_BLK, W64), lambda i: (0, i, 0)),
            pl.BlockSpec((NC, ROW_BLK, W64), lambda i: (0, i, 0)),
            pl.BlockSpec((ROW_BLK, W64), lambda i: (i, 0)),
            pl.BlockSpec((ROW_BLK, W64), lambda i: (i, 0)),
            pl.BlockSpec((ROW_BLK, NFEAT), lambda i: (i, 0)),
            pl.BlockSpec((NFEAT, NHID), lambda i: (0, 0)),
            pl.BlockSpec((1, NHID), lambda i: (0, 0)),
            pl.BlockSpec((NHID, NCLASS), lambda i: (0, 0)),
        ],
        out_specs=pl.BlockSpec((ROW_BLK, NCLASS), lambda i: (i, 0)),
        out_shape=jax.ShapeDtypeStruct((ACC_ROWS, NCLASS), jnp.float32),
    )(aggl, aggr, xpl, xpr, dinvb, W1, b1r, W2)


def _final_body(aggp_ref, y_ref, dinvb_ref, b2_ref, out_ref):
    ap = aggp_ref[...]
    dinv = dinvb_ref[...][:, :NCLASS]
    z = dinv * (ap[0] + ap[1] + y_ref[...]) + b2_ref[...]
    zs = z * 5.0
    m = jnp.max(zs, axis=1, keepdims=True)
    e = jnp.exp(zs - m)
    lse = jnp.log(jnp.sum(e, axis=1, keepdims=True))
    out_ref[...] = (zs - m) - lse


def _tc_final(aggp, y, dinvb, b2r):
    return pl.pallas_call(
        _final_body,
        grid=(N_BLOCKS,),
        in_specs=[
            pl.BlockSpec((NC, ROW_BLK, NCLASS), lambda i: (0, i, 0)),
            pl.BlockSpec((ROW_BLK, NCLASS), lambda i: (i, 0)),
            pl.BlockSpec((ROW_BLK, NFEAT), lambda i: (i, 0)),
            pl.BlockSpec((1, NCLASS), lambda i: (0, 0)),
        ],
        out_specs=pl.BlockSpec((ROW_BLK, NCLASS), lambda i: (i, 0)),
        out_shape=jax.ShapeDtypeStruct((ACC_ROWS, NCLASS), jnp.float32),
    )(aggp, y, dinvb, b2r)


def kernel(x, edge_index, W1, b1, W2, b2):
    src = edge_index[0].astype(jnp.int32)
    dst = edge_index[1].astype(jnp.int32)
    npad = EDGE_PAD - N_EDGES
    srcp = jnp.concatenate([src, jnp.zeros((npad,), jnp.int32)])
    dstp = jnp.concatenate([dst, jnp.full((npad,), N_NODES, jnp.int32)])
    srcf = srcp.reshape(EDGE_PAD // CHUNK, CHUNK)
    dstf = dstp.reshape(EDGE_PAD // CHUNK, CHUNK)
    dst3f = dstp.reshape(NC, NS, NCHUNK * CHUNK)

    zeros1 = jnp.zeros((ACC_ROWS,), jnp.float32)
    zeros64 = jnp.zeros((ACC_ROWS, W64), jnp.float32)

    xpad = jnp.zeros((ACC_ROWS, NFEAT), jnp.float32).at[:N_NODES].set(x)

    degp = _deg_pass(dst3f, zeros1)
    degp_t = degp.reshape(NC * NS, ACC_ROWS).T
    xpl, xpr, dinvb = _tc_prescale(degp_t, xpad)
    aggl, aggr = _agg_pass(srcf, dstf, [xpl, xpr], zeros64)
    y = _tc_mid(aggl, aggr, xpl, xpr, dinvb, W1, b1.reshape(1, NHID), W2)
    (agg2,) = _agg_pass(srcf, dstf, [y], zeros64)
    out = _tc_final(agg2, y, dinvb, b2.reshape(1, NCLASS))
    return out[:N_NODES]
